# baseline (speedup 1.0000x reference)
import functools

import jax
import jax.numpy as jnp
import numpy as np
from jax import lax
from jax.experimental import pallas as pl
from jax.experimental.pallas import tpu as pltpu
from jax.experimental.pallas import tpu_sc as plsc

_N = 10000
_E = 320000
_D = 128
_PERC = 0.5

_IDX = np.sort(
    np.random.default_rng(0).choice(_N, size=int(np.round(_N * _PERC)), replace=False)
).astype(np.int32)
_M = _IDX.shape[0]
_MP = 5008
_REMAP = np.full((_N,), _M, dtype=np.int32)
_REMAP[_IDX] = np.arange(_M, dtype=np.int32)

_NC, _NS, _L = 2, 16, 16
_NW = _NC * _NS
_FPW = _D // _NW
_C = 4000
_NCH = _E // _C


def _linear_body(x_ref, wt_ref, b_ref, o_ref):
    o_ref[...] = jnp.dot(x_ref[...], wt_ref[...], preferred_element_type=jnp.float32) + b_ref[...]


@jax.jit
def _linear(x, W, b):
    blk = 400
    grid = (x.shape[0] // blk,)
    return pl.pallas_call(
        _linear_body,
        grid=grid,
        in_specs=[
            pl.BlockSpec((blk, _D), lambda i: (i, 0)),
            pl.BlockSpec((_D, _D), lambda i: (0, 0)),
            pl.BlockSpec((1, _D), lambda i: (0, 0)),
        ],
        out_specs=pl.BlockSpec((blk, _D), lambda i: (i, 0)),
        out_shape=jax.ShapeDtypeStruct((x.shape[0], _D), jnp.float32),
    )(x, W.T, b[None, :])


_MESH = plsc.VectorSubcoreMesh(
    core_axis_name="c", subcore_axis_name="s", num_cores=_NC, num_subcores=_NS
)


@functools.partial(
    pl.kernel,
    out_type=jax.ShapeDtypeStruct((_D, _MP), jnp.float32),
    mesh=_MESH,
    compiler_params=pltpu.CompilerParams(needs_layout_passes=False),
    scratch_types=[
        pltpu.VMEM((_FPW, _N), jnp.float32),
        pltpu.VMEM((_FPW, _MP), jnp.float32),
        pltpu.VMEM((_N,), jnp.int32),
        pltpu.VMEM((_C,), jnp.int32),
        pltpu.VMEM((_C,), jnp.int32),
    ],
)
def _scmax(ht, rowh, colh, remaph, out, h_v, acc_v, remap_v, row_v, col_v):
    c = lax.axis_index("c")
    s = lax.axis_index("s")
    w = s * _NC + c

    pltpu.sync_copy(ht.at[pl.ds(w * _FPW, _FPW), :], h_v)
    pltpu.sync_copy(remaph, remap_v)

    neg = jnp.full((_L,), -jnp.inf, dtype=jnp.float32)
    for f in range(_FPW):
        def _init(i, carry, f=f):
            acc_v[f, pl.ds(i * _L, _L)] = neg
            return carry
        lax.fori_loop(0, _MP // _L, _init, 0)

    lane = lax.iota(jnp.int32, _L)
    e_of = lane >> 2
    f_of = lane & 3

    def _group(g, carry):
        eidx = e_of + g * 4
        rows = plsc.load_gather(row_v, [eidx])
        cols = plsc.load_gather(col_v, [eidx])
        mc = plsc.load_gather(remap_v, [cols])
        hv = plsc.load_gather(h_v, [f_of, rows])
        av = plsc.load_gather(acc_v, [f_of, mc])
        nv = jnp.maximum(av, hv)
        plsc.store_scatter(acc_v, [f_of, mc], nv)
        av2 = plsc.load_gather(acc_v, [f_of, mc])
        bad = hv > av2
        nbad = jnp.max(bad.astype(jnp.int32))

        @pl.when(nbad > 0)
        def _fix():
            b = bad
            for _ in range(3):
                a1 = plsc.load_gather(acc_v, [f_of, mc], mask=b)
                n1 = jnp.maximum(a1, hv)
                plsc.store_scatter(acc_v, [f_of, mc], n1, mask=b)
                a2 = plsc.load_gather(acc_v, [f_of, mc], mask=b)
                b = jnp.logical_and(b, hv > a2)

        return carry

    def _chunk(cix, carry):
        pltpu.sync_copy(rowh.at[pl.ds(cix * _C, _C)], row_v)
        pltpu.sync_copy(colh.at[pl.ds(cix * _C, _C)], col_v)
        lax.fori_loop(0, _C // 4, _group, 0)
        return carry

    lax.fori_loop(0, _NCH, _chunk, 0)

    for f in range(_FPW):
        def _drain(i, carry, f=f):
            v = acc_v[f, pl.ds(i * _L, _L)]
            acc_v[f, pl.ds(i * _L, _L)] = jnp.where(v == -jnp.inf, 0.0, v)
            return carry
        lax.fori_loop(0, _MP // _L, _drain, 0)
    pltpu.sync_copy(acc_v, out.at[pl.ds(w * _FPW, _FPW), :])


def kernel(x, pos, batch, edge_index, W, b):
    h = _linear(x.astype(jnp.float32), W, b)
    ht = h.T
    row, col = edge_index[0], edge_index[1]
    out_t = _scmax(ht, row, col, jnp.asarray(_REMAP))
    pooled = out_t[:, :_M].T
    idx = jnp.asarray(_IDX)
    return pooled, pos[idx], batch[idx]


# speedup vs baseline: 5.6505x; 5.6505x over previous
import functools

import jax
import jax.numpy as jnp
import numpy as np
from jax import lax
from jax.experimental import pallas as pl
from jax.experimental.pallas import tpu as pltpu
from jax.experimental.pallas import tpu_sc as plsc

_N = 10000
_E = 320000
_D = 128
_PERC = 0.5

_IDX = np.sort(
    np.random.default_rng(0).choice(_N, size=int(np.round(_N * _PERC)), replace=False)
).astype(np.int32)
_M = _IDX.shape[0]
_MP = 5008
_REMAP = np.full((_N,), _M, dtype=np.int32)
_REMAP[_IDX] = np.arange(_M, dtype=np.int32)

_NC, _NS, _L = 2, 16, 16
_NW = _NC * _NS
_FPW = _D // _NW
_C = 4000
_NCH = _E // _C


def _linear_body(x_ref, wt_ref, b_ref, o_ref):
    o_ref[...] = jnp.dot(x_ref[...], wt_ref[...], preferred_element_type=jnp.float32) + b_ref[...]


@jax.jit
def _linear(x, W, b):
    blk = 400
    grid = (x.shape[0] // blk,)
    return pl.pallas_call(
        _linear_body,
        grid=grid,
        in_specs=[
            pl.BlockSpec((blk, _D), lambda i: (i, 0)),
            pl.BlockSpec((_D, _D), lambda i: (0, 0)),
            pl.BlockSpec((1, _D), lambda i: (0, 0)),
        ],
        out_specs=pl.BlockSpec((blk, _D), lambda i: (i, 0)),
        out_shape=jax.ShapeDtypeStruct((x.shape[0], _D), jnp.float32),
    )(x, W.T, b[None, :])


_MESH = plsc.VectorSubcoreMesh(
    core_axis_name="c", subcore_axis_name="s", num_cores=_NC, num_subcores=_NS
)


@functools.partial(
    pl.kernel,
    out_type=jax.ShapeDtypeStruct((_D, _MP), jnp.float32),
    mesh=_MESH,
    compiler_params=pltpu.CompilerParams(needs_layout_passes=False),
    scratch_types=[
        pltpu.VMEM((_N,), jnp.float32),
        pltpu.VMEM((_N,), jnp.float32),
        pltpu.VMEM((_N,), jnp.float32),
        pltpu.VMEM((_N,), jnp.float32),
        pltpu.VMEM((_MP,), jnp.float32),
        pltpu.VMEM((_MP,), jnp.float32),
        pltpu.VMEM((_MP,), jnp.float32),
        pltpu.VMEM((_MP,), jnp.float32),
        pltpu.VMEM((_N,), jnp.int32),
        pltpu.VMEM((_C,), jnp.int32),
        pltpu.VMEM((_C,), jnp.int32),
    ],
)
def _scmax(ht, rowh, colh, remaph, out,
           h0, h1, h2, h3, a0, a1, a2, a3, remap_v, row_v, col_v):
    c = lax.axis_index("c")
    s = lax.axis_index("s")
    w = s * _NC + c
    hs = (h0, h1, h2, h3)
    accs = (a0, a1, a2, a3)

    for f in range(_FPW):
        pltpu.sync_copy(ht.at[w * _FPW + f], hs[f])
    pltpu.sync_copy(remaph, remap_v)

    neg = jnp.full((_L,), -jnp.inf, dtype=jnp.float32)
    for f in range(_FPW):
        def _init(i, carry, f=f):
            accs[f][pl.ds(i * _L, _L)] = neg
            return carry
        lax.fori_loop(0, _MP // _L, _init, 0)

    def _block(g, carry):
        rows = row_v[pl.ds(g * _L, _L)]
        cols = col_v[pl.ds(g * _L, _L)]
        mc = plsc.load_gather(remap_v, [cols])
        sel = mc < _M
        _, last = plsc.scan_count(mc, sel)
        lastsel = jnp.logical_and(last, sel)
        okst = jnp.logical_or(lastsel, jnp.logical_not(sel))
        hv = []
        for f in range(_FPW):
            hvf = plsc.load_gather(hs[f], [rows])
            avf = plsc.load_gather(accs[f], [mc])
            plsc.store_scatter(accs[f], [mc], jnp.maximum(avf, hvf), mask=okst)
            hv.append(hvf)
        rem = jnp.logical_and(sel, jnp.logical_not(lastsel))
        nrem = plsc.all_reduce_population_count(rem)[0]

        @pl.when(nrem > 0)
        def _fix():
            def _cond(st):
                r, i = st
                return jnp.logical_and(
                    plsc.all_reduce_population_count(r)[0] > 0, i < _L)

            def _round(st):
                r, i = st
                _, l2 = plsc.scan_count(mc, r)
                m2 = jnp.logical_and(l2, r)
                for f in range(_FPW):
                    avf = plsc.load_gather(accs[f], [mc], mask=m2)
                    plsc.store_scatter(
                        accs[f], [mc], jnp.maximum(avf, hv[f]), mask=m2)
                return jnp.logical_and(r, jnp.logical_not(m2)), i + 1

            lax.while_loop(_cond, _round, (rem, jnp.int32(0)))

        return carry

    def _chunk(cix, carry):
        pltpu.sync_copy(rowh.at[pl.ds(cix * _C, _C)], row_v)
        pltpu.sync_copy(colh.at[pl.ds(cix * _C, _C)], col_v)
        lax.fori_loop(0, _C // _L, _block, 0)
        return carry

    lax.fori_loop(0, _NCH, _chunk, 0)

    for f in range(_FPW):
        def _drain(i, carry, f=f):
            v = accs[f][pl.ds(i * _L, _L)]
            accs[f][pl.ds(i * _L, _L)] = jnp.where(v == -jnp.inf, 0.0, v)
            return carry
        lax.fori_loop(0, _MP // _L, _drain, 0)
        pltpu.sync_copy(accs[f], out.at[w * _FPW + f])


def kernel(x, pos, batch, edge_index, W, b):
    h = _linear(x.astype(jnp.float32), W, b)
    ht = h.T
    row, col = edge_index[0], edge_index[1]
    out_t = _scmax(ht, row, col, jnp.asarray(_REMAP))
    pooled = out_t[:, :_M].T
    idx = jnp.asarray(_IDX)
    return pooled, pos[idx], batch[idx]


# speedup vs baseline: 6.0262x; 1.0665x over previous
import functools

import jax
import jax.numpy as jnp
import numpy as np
from jax import lax
from jax.experimental import pallas as pl
from jax.experimental.pallas import tpu as pltpu
from jax.experimental.pallas import tpu_sc as plsc

_N = 10000
_E = 320000
_D = 128
_PERC = 0.5

_IDX = np.sort(
    np.random.default_rng(0).choice(_N, size=int(np.round(_N * _PERC)), replace=False)
).astype(np.int32)
_M = _IDX.shape[0]
_MP = 5008
_REMAP = np.full((_N,), _M, dtype=np.int32)
_REMAP[_IDX] = np.arange(_M, dtype=np.int32)

_NC, _NS, _L = 2, 16, 16
_NW = _NC * _NS
_FPW = _D // _NW
_C = 16000
_NCH = _E // _C


def _linear_body(x_ref, wt_ref, b_ref, o_ref):
    o_ref[...] = jnp.dot(x_ref[...], wt_ref[...], preferred_element_type=jnp.float32) + b_ref[...]


@jax.jit
def _linear(x, W, b):
    blk = 400
    grid = (x.shape[0] // blk,)
    return pl.pallas_call(
        _linear_body,
        grid=grid,
        in_specs=[
            pl.BlockSpec((blk, _D), lambda i: (i, 0)),
            pl.BlockSpec((_D, _D), lambda i: (0, 0)),
            pl.BlockSpec((1, _D), lambda i: (0, 0)),
        ],
        out_specs=pl.BlockSpec((blk, _D), lambda i: (i, 0)),
        out_shape=jax.ShapeDtypeStruct((x.shape[0], _D), jnp.float32),
    )(x, W.T, b[None, :])


_MESH = plsc.VectorSubcoreMesh(
    core_axis_name="c", subcore_axis_name="s", num_cores=_NC, num_subcores=_NS
)


@functools.partial(
    pl.kernel,
    out_type=jax.ShapeDtypeStruct((_D, _MP), jnp.float32),
    mesh=_MESH,
    compiler_params=pltpu.CompilerParams(needs_layout_passes=False),
    scratch_types=[
        pltpu.VMEM((_N,), jnp.float32),
        pltpu.VMEM((_N,), jnp.float32),
        pltpu.VMEM((_N,), jnp.float32),
        pltpu.VMEM((_N,), jnp.float32),
        pltpu.VMEM((_MP,), jnp.float32),
        pltpu.VMEM((_MP,), jnp.float32),
        pltpu.VMEM((_MP,), jnp.float32),
        pltpu.VMEM((_MP,), jnp.float32),
        pltpu.VMEM((_N,), jnp.int32),
        pltpu.VMEM((_C,), jnp.int32),
        pltpu.VMEM((_C,), jnp.int32),
    ],
)
def _scmax(ht, rowh, colh, remaph, out,
           h0, h1, h2, h3, a0, a1, a2, a3, remap_v, row_v, col_v):
    c = lax.axis_index("c")
    s = lax.axis_index("s")
    w = s * _NC + c
    hs = (h0, h1, h2, h3)
    accs = (a0, a1, a2, a3)

    for f in range(_FPW):
        pltpu.sync_copy(ht.at[w * _FPW + f], hs[f])
    pltpu.sync_copy(remaph, remap_v)

    neg = jnp.full((_L,), -jnp.inf, dtype=jnp.float32)
    for f in range(_FPW):
        def _init(i, carry, f=f):
            accs[f][pl.ds(i * _L, _L)] = neg
            return carry
        lax.fori_loop(0, _MP // _L, _init, 0)

    def _block(g, carry):
        rows = row_v[pl.ds(g * _L, _L)]
        cols = col_v[pl.ds(g * _L, _L)]
        mc = plsc.load_gather(remap_v, [cols])
        sel = mc < _M
        _, last = plsc.scan_count(mc, sel)
        lastsel = jnp.logical_and(last, sel)
        okst = jnp.logical_or(lastsel, jnp.logical_not(sel))
        hv = []
        for f in range(_FPW):
            hvf = plsc.load_gather(hs[f], [rows])
            avf = plsc.load_gather(accs[f], [mc])
            plsc.store_scatter(accs[f], [mc], jnp.maximum(avf, hvf), mask=okst)
            hv.append(hvf)
        rem = jnp.logical_and(sel, jnp.logical_not(lastsel))
        nrem = plsc.all_reduce_population_count(rem)[0]

        @pl.when(nrem > 0)
        def _fix():
            def _cond(st):
                r, i = st
                return jnp.logical_and(
                    plsc.all_reduce_population_count(r)[0] > 0, i < _L)

            def _round(st):
                r, i = st
                _, l2 = plsc.scan_count(mc, r)
                m2 = jnp.logical_and(l2, r)
                for f in range(_FPW):
                    avf = plsc.load_gather(accs[f], [mc], mask=m2)
                    plsc.store_scatter(
                        accs[f], [mc], jnp.maximum(avf, hv[f]), mask=m2)
                return jnp.logical_and(r, jnp.logical_not(m2)), i + 1

            lax.while_loop(_cond, _round, (rem, jnp.int32(0)))

        return carry

    def _chunk(cix, carry):
        pltpu.sync_copy(rowh.at[pl.ds(cix * _C, _C)], row_v)
        pltpu.sync_copy(colh.at[pl.ds(cix * _C, _C)], col_v)
        lax.fori_loop(0, _C // _L, _block, 0)
        return carry

    lax.fori_loop(0, _NCH, _chunk, 0)

    for f in range(_FPW):
        def _drain(i, carry, f=f):
            v = accs[f][pl.ds(i * _L, _L)]
            accs[f][pl.ds(i * _L, _L)] = jnp.where(v == -jnp.inf, 0.0, v)
            return carry
        lax.fori_loop(0, _MP // _L, _drain, 0)
        pltpu.sync_copy(accs[f], out.at[w * _FPW + f])


def kernel(x, pos, batch, edge_index, W, b):
    h = _linear(x.astype(jnp.float32), W, b)
    ht = h.T
    row, col = edge_index[0], edge_index[1]
    out_t = _scmax(ht, row, col, jnp.asarray(_REMAP))
    pooled = out_t[:, :_M].T
    idx = jnp.asarray(_IDX)
    return pooled, pos[idx], batch[idx]


# speedup vs baseline: 6.1424x; 1.0193x over previous
import functools

import jax
import jax.numpy as jnp
import numpy as np
from jax import lax
from jax.experimental import pallas as pl
from jax.experimental.pallas import tpu as pltpu
from jax.experimental.pallas import tpu_sc as plsc

_N = 10000
_E = 320000
_D = 128
_PERC = 0.5

_IDX = np.sort(
    np.random.default_rng(0).choice(_N, size=int(np.round(_N * _PERC)), replace=False)
).astype(np.int32)
_M = _IDX.shape[0]
_MP = 5008
_REMAP = np.full((_N,), _M, dtype=np.int32)
_REMAP[_IDX] = np.arange(_M, dtype=np.int32)

_NC, _NS, _L = 2, 16, 16
_NW = _NC * _NS
_FPW = _D // _NW
_C = 16000
_NCH = _E // _C


def _linear_body(x_ref, wt_ref, b_ref, o_ref):
    o_ref[...] = jnp.dot(x_ref[...], wt_ref[...], preferred_element_type=jnp.float32) + b_ref[...]


@jax.jit
def _linear(x, W, b):
    blk = 400
    grid = (x.shape[0] // blk,)
    return pl.pallas_call(
        _linear_body,
        grid=grid,
        in_specs=[
            pl.BlockSpec((blk, _D), lambda i: (i, 0)),
            pl.BlockSpec((_D, _D), lambda i: (0, 0)),
            pl.BlockSpec((1, _D), lambda i: (0, 0)),
        ],
        out_specs=pl.BlockSpec((blk, _D), lambda i: (i, 0)),
        out_shape=jax.ShapeDtypeStruct((x.shape[0], _D), jnp.float32),
    )(x, W.T, b[None, :])


_MESH = plsc.VectorSubcoreMesh(
    core_axis_name="c", subcore_axis_name="s", num_cores=_NC, num_subcores=_NS
)


@functools.partial(
    pl.kernel,
    out_type=jax.ShapeDtypeStruct((_D, _MP), jnp.float32),
    mesh=_MESH,
    compiler_params=pltpu.CompilerParams(needs_layout_passes=False),
    scratch_types=[
        pltpu.VMEM((_N,), jnp.float32),
        pltpu.VMEM((_N,), jnp.float32),
        pltpu.VMEM((_N,), jnp.float32),
        pltpu.VMEM((_N,), jnp.float32),
        pltpu.VMEM((_MP,), jnp.float32),
        pltpu.VMEM((_MP,), jnp.float32),
        pltpu.VMEM((_MP,), jnp.float32),
        pltpu.VMEM((_MP,), jnp.float32),
        pltpu.VMEM((_MP,), jnp.float32),
        pltpu.VMEM((_MP,), jnp.float32),
        pltpu.VMEM((_MP,), jnp.float32),
        pltpu.VMEM((_MP,), jnp.float32),
        pltpu.VMEM((_N,), jnp.int32),
        pltpu.VMEM((_C,), jnp.int32),
        pltpu.VMEM((_C,), jnp.int32),
    ],
)
def _scmax(ht, rowh, colh, remaph, out,
           h0, h1, h2, h3, a0, a1, a2, a3, b0, b1, b2, b3,
           remap_v, row_v, col_v):
    c = lax.axis_index("c")
    s = lax.axis_index("s")
    w = s * _NC + c
    hs = (h0, h1, h2, h3)
    accsA = (a0, a1, a2, a3)
    accsB = (b0, b1, b2, b3)

    for f in range(_FPW):
        pltpu.sync_copy(ht.at[w * _FPW + f], hs[f])
    pltpu.sync_copy(remaph, remap_v)

    neg = jnp.full((_L,), -jnp.inf, dtype=jnp.float32)
    for accs in (accsA, accsB):
        for f in range(_FPW):
            def _init(i, carry, f=f, accs=accs):
                accs[f][pl.ds(i * _L, _L)] = neg
                return carry
            lax.fori_loop(0, _MP // _L, _init, 0)

    def _do_block(g, accs):
        rows = row_v[pl.ds(g * _L, _L)]
        cols = col_v[pl.ds(g * _L, _L)]
        mc = plsc.load_gather(remap_v, [cols])
        sel = mc < _M
        _, last = plsc.scan_count(mc, sel)
        lastsel = jnp.logical_and(last, sel)
        okst = jnp.logical_or(lastsel, jnp.logical_not(sel))
        hv = []
        for f in range(_FPW):
            hvf = plsc.load_gather(hs[f], [rows])
            avf = plsc.load_gather(accs[f], [mc])
            plsc.store_scatter(accs[f], [mc], jnp.maximum(avf, hvf), mask=okst)
            hv.append(hvf)
        rem = jnp.logical_and(sel, jnp.logical_not(lastsel))
        nrem = plsc.all_reduce_population_count(rem)[0]

        @pl.when(nrem > 0)
        def _fix():
            def _cond(st):
                r, i = st
                return jnp.logical_and(
                    plsc.all_reduce_population_count(r)[0] > 0, i < _L)

            def _round(st):
                r, i = st
                _, l2 = plsc.scan_count(mc, r)
                m2 = jnp.logical_and(l2, r)
                for f in range(_FPW):
                    avf = plsc.load_gather(accs[f], [mc], mask=m2)
                    plsc.store_scatter(
                        accs[f], [mc], jnp.maximum(avf, hv[f]), mask=m2)
                return jnp.logical_and(r, jnp.logical_not(m2)), i + 1

            lax.while_loop(_cond, _round, (rem, jnp.int32(0)))

    _NBH = _C // _L // 2

    def _block(g, carry):
        _do_block(g, accsA)
        _do_block(g + _NBH, accsB)
        return carry

    def _chunk(cix, carry):
        pltpu.sync_copy(rowh.at[pl.ds(cix * _C, _C)], row_v)
        pltpu.sync_copy(colh.at[pl.ds(cix * _C, _C)], col_v)
        lax.fori_loop(0, _NBH, _block, 0)
        return carry

    lax.fori_loop(0, _NCH, _chunk, 0)

    for f in range(_FPW):
        def _drain(i, carry, f=f):
            v = jnp.maximum(accsA[f][pl.ds(i * _L, _L)],
                            accsB[f][pl.ds(i * _L, _L)])
            accsA[f][pl.ds(i * _L, _L)] = jnp.where(v == -jnp.inf, 0.0, v)
            return carry
        lax.fori_loop(0, _MP // _L, _drain, 0)
        pltpu.sync_copy(accsA[f], out.at[w * _FPW + f])


def kernel(x, pos, batch, edge_index, W, b):
    h = _linear(x.astype(jnp.float32), W, b)
    ht = h.T
    row, col = edge_index[0], edge_index[1]
    out_t = _scmax(ht, row, col, jnp.asarray(_REMAP))
    pooled = out_t[:, :_M].T
    idx = jnp.asarray(_IDX)
    return pooled, pos[idx], batch[idx]


# speedup vs baseline: 11.4587x; 1.8655x over previous
import functools

import jax
import jax.numpy as jnp
import numpy as np
from jax import lax
from jax.experimental import pallas as pl
from jax.experimental.pallas import tpu as pltpu
from jax.experimental.pallas import tpu_sc as plsc

_N = 10000
_E = 320000
_D = 128
_PERC = 0.5

_IDX = np.sort(
    np.random.default_rng(0).choice(_N, size=int(np.round(_N * _PERC)), replace=False)
).astype(np.int32)
_M = _IDX.shape[0]
_MP = 5008
_REMAP = np.full((_N,), _M, dtype=np.int32)
_REMAP[_IDX] = np.arange(_M, dtype=np.int32)

_NC, _NS, _L = 2, 16, 16
_NW = _NC * _NS
_FPW = _D // _NW
_EPW = _E // _NW
_CAP = 10400


def _linear_body(x_ref, wt_ref, b_ref, o_ref):
    o_ref[...] = jnp.dot(x_ref[...], wt_ref[...], preferred_element_type=jnp.float32) + b_ref[...]


@jax.jit
def _linear(x, W, b):
    blk = 400
    grid = (x.shape[0] // blk,)
    return pl.pallas_call(
        _linear_body,
        grid=grid,
        in_specs=[
            pl.BlockSpec((blk, _D), lambda i: (i, 0)),
            pl.BlockSpec((_D, _D), lambda i: (0, 0)),
            pl.BlockSpec((1, _D), lambda i: (0, 0)),
        ],
        out_specs=pl.BlockSpec((blk, _D), lambda i: (i, 0)),
        out_shape=jax.ShapeDtypeStruct((x.shape[0], _D), jnp.float32),
    )(x, W.T, b[None, :])


_MESH = plsc.VectorSubcoreMesh(
    core_axis_name="c", subcore_axis_name="s", num_cores=_NC, num_subcores=_NS
)
_CPARAMS = pltpu.CompilerParams(needs_layout_passes=False)


@functools.partial(
    pl.kernel,
    out_type=(
        jax.ShapeDtypeStruct((_NW * _CAP,), jnp.int32),
        jax.ShapeDtypeStruct((_NW * _CAP,), jnp.int32),
        jax.ShapeDtypeStruct((_NW, _L), jnp.int32),
    ),
    mesh=_MESH,
    compiler_params=_CPARAMS,
    scratch_types=[
        pltpu.VMEM((_EPW,), jnp.int32),
        pltpu.VMEM((_EPW,), jnp.int32),
        pltpu.VMEM((_CAP,), jnp.int32),
        pltpu.VMEM((_CAP,), jnp.int32),
        pltpu.VMEM((_N,), jnp.int32),
        pltpu.VMEM((_L,), jnp.int32),
    ],
)
def _compact(rowh, colh, remaph, crowh, cmch, cnth,
             row_v, col_v, crow_v, cmc_v, remap_v, cnt_v):
    c = lax.axis_index("c")
    s = lax.axis_index("s")
    w = s * _NC + c

    pltpu.sync_copy(rowh.at[pl.ds(w * _EPW, _EPW)], row_v)
    pltpu.sync_copy(colh.at[pl.ds(w * _EPW, _EPW)], col_v)
    pltpu.sync_copy(remaph, remap_v)
    lane = lax.iota(jnp.int32, _L)

    def _blk(g, cnt):
        rows = row_v[pl.ds(g * _L, _L)]
        cols = col_v[pl.ds(g * _L, _L)]
        mc = plsc.load_gather(remap_v, [cols])
        sel = mc < _M
        tgt = plsc.cumsum(sel.astype(jnp.int32)) + (cnt - 1)
        plsc.store_scatter(crow_v, [tgt], rows, mask=sel)
        plsc.store_scatter(cmc_v, [tgt], mc, mask=sel)
        return cnt + plsc.all_reduce_population_count(sel)[0]

    cnt = lax.fori_loop(0, _EPW // _L, _blk, jnp.int32(0))

    plsc.store_scatter(crow_v, [cnt + lane], jnp.zeros((_L,), jnp.int32))
    plsc.store_scatter(cmc_v, [cnt + lane], jnp.full((_L,), _M, jnp.int32))
    nb = (cnt + _L - 1) // _L
    cnt_v[...] = jnp.broadcast_to(nb, (_L,))

    pltpu.sync_copy(crow_v, crowh.at[pl.ds(w * _CAP, _CAP)])
    pltpu.sync_copy(cmc_v, cmch.at[pl.ds(w * _CAP, _CAP)])
    pltpu.sync_copy(cnt_v, cnth.at[w])


@functools.partial(
    pl.kernel,
    out_type=jax.ShapeDtypeStruct((_D, _MP), jnp.float32),
    mesh=_MESH,
    compiler_params=_CPARAMS,
    scratch_types=[
        pltpu.VMEM((_N,), jnp.float32),
        pltpu.VMEM((_N,), jnp.float32),
        pltpu.VMEM((_N,), jnp.float32),
        pltpu.VMEM((_N,), jnp.float32),
        pltpu.VMEM((_MP,), jnp.float32),
        pltpu.VMEM((_MP,), jnp.float32),
        pltpu.VMEM((_MP,), jnp.float32),
        pltpu.VMEM((_MP,), jnp.float32),
        pltpu.VMEM((_CAP,), jnp.int32),
        pltpu.VMEM((_CAP,), jnp.int32),
        pltpu.VMEM((_NW, _L), jnp.int32),
    ],
)
def _scmax(ht, crowh, cmch, cnth, out,
           h0, h1, h2, h3, a0, a1, a2, a3, rows_v, mc_v, cnt_v):
    c = lax.axis_index("c")
    s = lax.axis_index("s")
    w = s * _NC + c
    hs = (h0, h1, h2, h3)
    accs = (a0, a1, a2, a3)

    for f in range(_FPW):
        pltpu.sync_copy(ht.at[w * _FPW + f], hs[f])
    pltpu.sync_copy(cnth, cnt_v)

    neg = jnp.full((_L,), -jnp.inf, dtype=jnp.float32)
    for f in range(_FPW):
        def _init(i, carry, f=f):
            accs[f][pl.ds(i * _L, _L)] = neg
            return carry
        lax.fori_loop(0, _MP // _L, _init, 0)

    def _block(g, carry):
        rows = rows_v[pl.ds(g * _L, _L)]
        mc = mc_v[pl.ds(g * _L, _L)]
        sel = mc < _M
        _, last = plsc.scan_count(mc, sel)
        lastsel = jnp.logical_and(last, sel)
        okst = jnp.logical_or(lastsel, jnp.logical_not(sel))
        hv = []
        for f in range(_FPW):
            hvf = plsc.load_gather(hs[f], [rows])
            avf = plsc.load_gather(accs[f], [mc])
            plsc.store_scatter(accs[f], [mc], jnp.maximum(avf, hvf), mask=okst)
            hv.append(hvf)
        rem = jnp.logical_and(sel, jnp.logical_not(lastsel))
        nrem = plsc.all_reduce_population_count(rem)[0]

        @pl.when(nrem > 0)
        def _fix():
            def _cond(st):
                r, i = st
                return jnp.logical_and(
                    plsc.all_reduce_population_count(r)[0] > 0, i < _L)

            def _round(st):
                r, i = st
                _, l2 = plsc.scan_count(mc, r)
                m2 = jnp.logical_and(l2, r)
                for f in range(_FPW):
                    avf = plsc.load_gather(accs[f], [mc], mask=m2)
                    plsc.store_scatter(
                        accs[f], [mc], jnp.maximum(avf, hv[f]), mask=m2)
                return jnp.logical_and(r, jnp.logical_not(m2)), i + 1

            lax.while_loop(_cond, _round, (rem, jnp.int32(0)))

        return carry

    for t in range(_NW):
        nb = cnt_v[t][0]

        @pl.when(nb > 0)
        def _slot(t=t, nb=nb):
            pltpu.sync_copy(crowh.at[pl.ds(t * _CAP, _CAP)], rows_v)
            pltpu.sync_copy(cmch.at[pl.ds(t * _CAP, _CAP)], mc_v)
            lax.fori_loop(0, nb, _block, 0)

    for f in range(_FPW):
        def _drain(i, carry, f=f):
            v = accs[f][pl.ds(i * _L, _L)]
            accs[f][pl.ds(i * _L, _L)] = jnp.where(v == -jnp.inf, 0.0, v)
            return carry
        lax.fori_loop(0, _MP // _L, _drain, 0)
        pltpu.sync_copy(accs[f], out.at[w * _FPW + f])


def kernel(x, pos, batch, edge_index, W, b):
    h = _linear(x.astype(jnp.float32), W, b)
    ht = h.T
    row, col = edge_index[0], edge_index[1]
    crow, cmc, cnts = _compact(row, col, jnp.asarray(_REMAP))
    out_t = _scmax(ht, crow, cmc, cnts)
    pooled = out_t[:, :_M].T
    idx = jnp.asarray(_IDX)
    return pooled, pos[idx], batch[idx]


# speedup vs baseline: 12.5884x; 1.0986x over previous
import functools

import jax
import jax.numpy as jnp
import numpy as np
from jax import lax
from jax.experimental import pallas as pl
from jax.experimental.pallas import tpu as pltpu
from jax.experimental.pallas import tpu_sc as plsc

_N = 10000
_E = 320000
_D = 128
_PERC = 0.5

_IDX = np.sort(
    np.random.default_rng(0).choice(_N, size=int(np.round(_N * _PERC)), replace=False)
).astype(np.int32)
_M = _IDX.shape[0]
_MP = 5008
_REMAP = np.full((_N,), _M, dtype=np.int32)
_REMAP[_IDX] = np.arange(_M, dtype=np.int32)

_NC, _NS, _L = 2, 16, 16
_NW = _NC * _NS
_FPW = _D // _NW
_EPW = _E // _NW
_CAP = 10400


def _linear_body(x_ref, wt_ref, b_ref, o_ref):
    o_ref[...] = jnp.dot(x_ref[...], wt_ref[...], preferred_element_type=jnp.float32) + b_ref[...]


@jax.jit
def _linear(x, W, b):
    blk = 400
    grid = (x.shape[0] // blk,)
    return pl.pallas_call(
        _linear_body,
        grid=grid,
        in_specs=[
            pl.BlockSpec((blk, _D), lambda i: (i, 0)),
            pl.BlockSpec((_D, _D), lambda i: (0, 0)),
            pl.BlockSpec((1, _D), lambda i: (0, 0)),
        ],
        out_specs=pl.BlockSpec((blk, _D), lambda i: (i, 0)),
        out_shape=jax.ShapeDtypeStruct((x.shape[0], _D), jnp.float32),
    )(x, W.T, b[None, :])


_MESH = plsc.VectorSubcoreMesh(
    core_axis_name="c", subcore_axis_name="s", num_cores=_NC, num_subcores=_NS
)
_CPARAMS = pltpu.CompilerParams(needs_layout_passes=False)


@functools.partial(
    pl.kernel,
    out_type=(
        jax.ShapeDtypeStruct((_NW * _CAP,), jnp.int32),
        jax.ShapeDtypeStruct((_NW * _CAP,), jnp.int32),
        jax.ShapeDtypeStruct((_NW, _L), jnp.int32),
        jax.ShapeDtypeStruct((_NW * _CAP,), jnp.int32),
        jax.ShapeDtypeStruct((_NW * _CAP,), jnp.int32),
        jax.ShapeDtypeStruct((_NW, _L), jnp.int32),
    ),
    mesh=_MESH,
    compiler_params=_CPARAMS,
    scratch_types=[
        pltpu.VMEM((_EPW,), jnp.int32),
        pltpu.VMEM((_EPW,), jnp.int32),
        pltpu.VMEM((_CAP,), jnp.int32),
        pltpu.VMEM((_CAP,), jnp.int32),
        pltpu.VMEM((_CAP,), jnp.int32),
        pltpu.VMEM((_CAP,), jnp.int32),
        pltpu.VMEM((_N,), jnp.int32),
        pltpu.VMEM((_L,), jnp.int32),
        pltpu.VMEM((_L,), jnp.int32),
    ],
)
def _compact(rowh, colh, remaph, crowh, cmch, cnth, srowh, smch, scnth,
             row_v, col_v, crow_v, cmc_v, srow_v, smc_v, remap_v, cnt_v, scnt_v):
    c = lax.axis_index("c")
    s = lax.axis_index("s")
    w = s * _NC + c

    pltpu.sync_copy(rowh.at[pl.ds(w * _EPW, _EPW)], row_v)
    pltpu.sync_copy(colh.at[pl.ds(w * _EPW, _EPW)], col_v)
    pltpu.sync_copy(remaph, remap_v)
    lane = lax.iota(jnp.int32, _L)

    def _blk(g, cnt):
        rows = row_v[pl.ds(g * _L, _L)]
        cols = col_v[pl.ds(g * _L, _L)]
        mc = plsc.load_gather(remap_v, [cols])
        sel = mc < _M
        tgt = plsc.cumsum(sel.astype(jnp.int32)) + (cnt - 1)
        plsc.store_scatter(crow_v, [tgt], rows, mask=sel)
        plsc.store_scatter(cmc_v, [tgt], mc, mask=sel)
        return cnt + plsc.all_reduce_population_count(sel)[0]

    cnt = lax.fori_loop(0, _EPW // _L, _blk, jnp.int32(0))

    plsc.store_scatter(crow_v, [cnt + lane], jnp.zeros((_L,), jnp.int32))
    plsc.store_scatter(cmc_v, [cnt + lane], jnp.full((_L,), _M, jnp.int32))
    nb = (cnt + _L - 1) // _L
    cnt_v[...] = jnp.broadcast_to(nb, (_L,))

    def _ana(g, scnt):
        mcb = cmc_v[pl.ds(g * _L, _L)]
        rowsb = crow_v[pl.ds(g * _L, _L)]
        sel = mcb < _M
        _, last = plsc.scan_count(mcb, sel)
        lastsel = jnp.logical_and(last, sel)
        okst = jnp.logical_or(lastsel, jnp.logical_not(sel))
        cmc_v[pl.ds(g * _L, _L)] = mcb | (okst.astype(jnp.int32) << 31)
        rem = jnp.logical_and(sel, jnp.logical_not(lastsel))
        tgt = plsc.cumsum(rem.astype(jnp.int32)) + (scnt - 1)
        plsc.store_scatter(srow_v, [tgt], rowsb, mask=rem)
        plsc.store_scatter(smc_v, [tgt], mcb, mask=rem)
        return scnt + plsc.all_reduce_population_count(rem)[0]

    scnt = lax.fori_loop(0, nb, _ana, jnp.int32(0))

    plsc.store_scatter(srow_v, [scnt + lane], jnp.zeros((_L,), jnp.int32))
    plsc.store_scatter(smc_v, [scnt + lane], jnp.full((_L,), _M, jnp.int32))
    snb = (scnt + _L - 1) // _L
    scnt_v[...] = jnp.broadcast_to(snb, (_L,))

    pltpu.sync_copy(crow_v, crowh.at[pl.ds(w * _CAP, _CAP)])
    pltpu.sync_copy(cmc_v, cmch.at[pl.ds(w * _CAP, _CAP)])
    pltpu.sync_copy(cnt_v, cnth.at[w])
    pltpu.sync_copy(srow_v, srowh.at[pl.ds(w * _CAP, _CAP)])
    pltpu.sync_copy(smc_v, smch.at[pl.ds(w * _CAP, _CAP)])
    pltpu.sync_copy(scnt_v, scnth.at[w])


@functools.partial(
    pl.kernel,
    out_type=jax.ShapeDtypeStruct((_D, _MP), jnp.float32),
    mesh=_MESH,
    compiler_params=_CPARAMS,
    scratch_types=[
        pltpu.VMEM((_N,), jnp.float32),
        pltpu.VMEM((_N,), jnp.float32),
        pltpu.VMEM((_N,), jnp.float32),
        pltpu.VMEM((_N,), jnp.float32),
        pltpu.VMEM((_MP,), jnp.float32),
        pltpu.VMEM((_MP,), jnp.float32),
        pltpu.VMEM((_MP,), jnp.float32),
        pltpu.VMEM((_MP,), jnp.float32),
        pltpu.VMEM((_CAP,), jnp.int32),
        pltpu.VMEM((_CAP,), jnp.int32),
        pltpu.VMEM((_NW, _L), jnp.int32),
        pltpu.VMEM((_NW, _L), jnp.int32),
    ],
)
def _scmax(ht, crowh, cmch, cnth, srowh, smch, scnth, out,
           h0, h1, h2, h3, a0, a1, a2, a3, rows_v, mc_v, cnt_v, scnt_v):
    c = lax.axis_index("c")
    s = lax.axis_index("s")
    w = s * _NC + c
    hs = (h0, h1, h2, h3)
    accs = (a0, a1, a2, a3)

    for f in range(_FPW):
        pltpu.sync_copy(ht.at[w * _FPW + f], hs[f])
    pltpu.sync_copy(cnth, cnt_v)
    pltpu.sync_copy(scnth, scnt_v)

    neg = jnp.full((_L,), -jnp.inf, dtype=jnp.float32)
    for f in range(_FPW):
        def _init(i, carry, f=f):
            accs[f][pl.ds(i * _L, _L)] = neg
            return carry
        lax.fori_loop(0, _MP // _L, _init, 0)

    def _main_block(g, carry):
        rows = rows_v[pl.ds(g * _L, _L)]
        mcm = mc_v[pl.ds(g * _L, _L)]
        okst = mcm < 0
        mc = mcm & jnp.int32(0x7FFFFFFF)
        for f in range(_FPW):
            hvf = plsc.load_gather(hs[f], [rows])
            avf = plsc.load_gather(accs[f], [mc])
            plsc.store_scatter(accs[f], [mc], jnp.maximum(avf, hvf), mask=okst)
        return carry

    def _spill_block(g, carry):
        rows = rows_v[pl.ds(g * _L, _L)]
        mc = mc_v[pl.ds(g * _L, _L)]
        sel = mc < _M
        _, last = plsc.scan_count(mc, sel)
        lastsel = jnp.logical_and(last, sel)
        okst = jnp.logical_or(lastsel, jnp.logical_not(sel))
        hv = []
        for f in range(_FPW):
            hvf = plsc.load_gather(hs[f], [rows])
            avf = plsc.load_gather(accs[f], [mc])
            plsc.store_scatter(accs[f], [mc], jnp.maximum(avf, hvf), mask=okst)
            hv.append(hvf)
        rem = jnp.logical_and(sel, jnp.logical_not(lastsel))
        nrem = plsc.all_reduce_population_count(rem)[0]

        @pl.when(nrem > 0)
        def _fix():
            def _cond(st):
                r, i = st
                return jnp.logical_and(
                    plsc.all_reduce_population_count(r)[0] > 0, i < _L)

            def _round(st):
                r, i = st
                _, l2 = plsc.scan_count(mc, r)
                m2 = jnp.logical_and(l2, r)
                for f in range(_FPW):
                    avf = plsc.load_gather(accs[f], [mc], mask=m2)
                    plsc.store_scatter(
                        accs[f], [mc], jnp.maximum(avf, hv[f]), mask=m2)
                return jnp.logical_and(r, jnp.logical_not(m2)), i + 1

            lax.while_loop(_cond, _round, (rem, jnp.int32(0)))

        return carry

    for t in range(_NW):
        nb = cnt_v[t][0]

        @pl.when(nb > 0)
        def _slot(t=t, nb=nb):
            pltpu.sync_copy(crowh.at[pl.ds(t * _CAP, _CAP)], rows_v)
            pltpu.sync_copy(cmch.at[pl.ds(t * _CAP, _CAP)], mc_v)
            lax.fori_loop(0, nb, _main_block, 0)

    for t in range(_NW):
        snb = scnt_v[t][0]

        @pl.when(snb > 0)
        def _spill_slot(t=t, snb=snb):
            pltpu.sync_copy(srowh.at[pl.ds(t * _CAP, _CAP)], rows_v)
            pltpu.sync_copy(smch.at[pl.ds(t * _CAP, _CAP)], mc_v)
            lax.fori_loop(0, snb, _spill_block, 0)

    for f in range(_FPW):
        def _drain(i, carry, f=f):
            v = accs[f][pl.ds(i * _L, _L)]
            accs[f][pl.ds(i * _L, _L)] = jnp.where(v == -jnp.inf, 0.0, v)
            return carry
        lax.fori_loop(0, _MP // _L, _drain, 0)
        pltpu.sync_copy(accs[f], out.at[w * _FPW + f])


def kernel(x, pos, batch, edge_index, W, b):
    h = _linear(x.astype(jnp.float32), W, b)
    ht = h.T
    row, col = edge_index[0], edge_index[1]
    crow, cmc, cnts, srow, smc, scnts = _compact(row, col, jnp.asarray(_REMAP))
    out_t = _scmax(ht, crow, cmc, cnts, srow, smc, scnts)
    pooled = out_t[:, :_M].T
    idx = jnp.asarray(_IDX)
    return pooled, pos[idx], batch[idx]
